# Initial kernel scaffold; baseline (speedup 1.0000x reference)
import jax
import jax.numpy as jnp
from jax.experimental import pallas as pl


def kernel(atom_feature, edge_index, bond_feature, distance, node2graph, b_factor, Wn, bn, We, be, Wm, bm, Wu, bu, Wo, bo):
    raise NotImplementedError("write your pallas kernel here")


# speedup vs baseline: 1.9437x; 1.9437x over previous
import functools

import jax
import jax.numpy as jnp
from jax import lax
from jax.experimental import pallas as pl
from jax.experimental.pallas import tpu as pltpu
from jax.experimental.pallas import tpu_sc as plsc

N = 10000
E = 320000
D_IN = 142
D_E_RAW = 5
NUM_ENC = 10
D_E = 25
DH = 128
NB = 5

NC = 2
NS = 16
L = 16
NW = NC * NS

CHUNK = 128
CPW = -(-E // (NW * CHUNK))
E_PAD = NW * CPW * CHUNK
N_PAD = 10240
ROWS_PER_TILE = N_PAD // NS


def _edge_pre_body(bond_ref, dist_ref, We_ref, be_ref, Wb_ref, bm_ref, *out_refs):
    k = lax.broadcasted_iota(jnp.int32, (1, NUM_ENC), 1).astype(jnp.float32)
    xs = dist_ref[...] * jnp.exp2(-k)
    e = jnp.concatenate([bond_ref[...], jnp.sin(xs), jnp.cos(xs)], axis=-1)
    eh = jnp.maximum(
        jnp.dot(e, We_ref[...], preferred_element_type=jnp.float32) + be_ref[...],
        0.0)
    for b in range(NB):
        out_refs[b][...] = (
            jnp.dot(eh, Wb_ref[b], preferred_element_type=jnp.float32)
            + bm_ref[b])


def _edge_pre(bond_p, dist_p, We, be2, Wm_bot, bm3):
    TE = 2048
    grid = (E_PAD // TE,)
    return pl.pallas_call(
        _edge_pre_body,
        grid=grid,
        in_specs=[
            pl.BlockSpec((TE, D_E_RAW), lambda i: (i, 0)),
            pl.BlockSpec((TE, 1), lambda i: (i, 0)),
            pl.BlockSpec((D_E, DH), lambda i: (0, 0)),
            pl.BlockSpec((1, DH), lambda i: (0, 0)),
            pl.BlockSpec((NB, DH, DH), lambda i: (0, 0, 0)),
            pl.BlockSpec((NB, 1, DH), lambda i: (0, 0, 0)),
        ],
        out_specs=[pl.BlockSpec((TE, DH), lambda i: (i, 0)) for _ in range(NB)],
        out_shape=[jax.ShapeDtypeStruct((E_PAD, DH), jnp.float32)
                   for _ in range(NB)],
    )(bond_p, dist_p, We, be2, Wm_bot, bm3)


def _node_init_body(x_ref, Wn_ref, bn_ref, Wt_ref, h_ref, hw_ref):
    h = jnp.maximum(
        jnp.dot(x_ref[...], Wn_ref[...], preferred_element_type=jnp.float32)
        + bn_ref[...], 0.0)
    h_ref[...] = h
    hw_ref[...] = jnp.dot(h, Wt_ref[...], preferred_element_type=jnp.float32)


def _node_init(atom, Wn, bn2, Wt0):
    return pl.pallas_call(
        _node_init_body,
        out_shape=[jax.ShapeDtypeStruct((N, DH), jnp.float32),
                   jax.ShapeDtypeStruct((N, DH), jnp.float32)],
    )(atom, Wn, bn2, Wt0)


def _update_body(h_ref, agg_ref, Wut_ref, Wub_ref, bu_ref, Wt_ref,
                 h_out_ref, hw_out_ref):
    agg = agg_ref[0, 0:N, :] + agg_ref[1, 0:N, :]
    hn = jnp.maximum(
        jnp.dot(h_ref[...], Wut_ref[...], preferred_element_type=jnp.float32)
        + jnp.dot(agg, Wub_ref[...], preferred_element_type=jnp.float32)
        + bu_ref[...], 0.0)
    h_out_ref[...] = hn
    hw_out_ref[...] = jnp.dot(hn, Wt_ref[...], preferred_element_type=jnp.float32)


def _update(h, agg2, Wut, Wub, bu2, Wt_next):
    return pl.pallas_call(
        _update_body,
        out_shape=[jax.ShapeDtypeStruct((N, DH), jnp.float32),
                   jax.ShapeDtypeStruct((N, DH), jnp.float32)],
    )(h, agg2, Wut, Wub, bu2, Wt_next)


def _update_last_body(h_ref, agg_ref, Wut_ref, Wub_ref, bu_ref, Wo_ref, bo_ref,
                      out_ref):
    agg = agg_ref[0, 0:N, :] + agg_ref[1, 0:N, :]
    hn = jnp.maximum(
        jnp.dot(h_ref[...], Wut_ref[...], preferred_element_type=jnp.float32)
        + jnp.dot(agg, Wub_ref[...], preferred_element_type=jnp.float32)
        + bu_ref[...], 0.0)
    logit = jnp.dot(hn, Wo_ref[...], preferred_element_type=jnp.float32) + bo_ref[...]
    out_ref[...] = 1.0 / (1.0 + jnp.exp(-logit))


def _update_last(h, agg2, Wut, Wub, bu2, Wo, bo2):
    return pl.pallas_call(
        _update_last_body,
        out_shape=jax.ShapeDtypeStruct((N, 1), jnp.float32),
    )(h, agg2, Wut, Wub, bu2, Wo, bo2)


def _sc_agg(hW, eWb, src_p, dst_p):
    mesh = plsc.VectorSubcoreMesh(core_axis_name="c", subcore_axis_name="s")

    @functools.partial(
        pl.kernel,
        out_type=jax.ShapeDtypeStruct((NC, N_PAD, DH), jnp.float32),
        mesh=mesh,
        scratch_types=[
            pltpu.VMEM((CHUNK,), jnp.int32),
            pltpu.VMEM((CHUNK,), jnp.int32),
            pltpu.VMEM((CHUNK, DH), jnp.float32),
            pltpu.VMEM((CHUNK, DH), jnp.float32),
            pltpu.VMEM_SHARED((N_PAD, DH), jnp.float32),
            pltpu.SemaphoreType.DMA,
        ],
    )
    def k(hW_hbm, eW_hbm, src_hbm, dst_hbm, out_hbm,
          sidx, didx, gat, ew, agg_sh, sem):
        cid = lax.axis_index("c")
        sid = lax.axis_index("s")
        wid = sid * NC + cid

        zero = jnp.zeros((L,), jnp.float32)

        @pl.loop(0, CHUNK)
        def _zero_rows(r):
            for j in range(DH // L):
                ew[r, pl.ds(j * L, L)] = zero

        for t in range(ROWS_PER_TILE // CHUNK):
            pltpu.sync_copy(
                ew, agg_sh.at[pl.ds(sid * ROWS_PER_TILE + t * CHUNK, CHUNK)])
        plsc.subcore_barrier()

        @pl.loop(0, CPW)
        def _chunk(c):
            base = (wid * CPW + c) * CHUNK
            pltpu.sync_copy(src_hbm.at[pl.ds(base, CHUNK)], sidx)
            pltpu.sync_copy(dst_hbm.at[pl.ds(base, CHUNK)], didx)
            pltpu.async_copy(hW_hbm.at[sidx], gat, sem).wait()
            pltpu.sync_copy(eW_hbm.at[pl.ds(base, CHUNK)], ew)

            @pl.loop(0, CHUNK)
            def _row(r):
                for j in range(DH // L):
                    s = pl.ds(j * L, L)
                    gat[r, s] = jnp.maximum(gat[r, s] + ew[r, s], 0.0)

            pltpu.sync_copy(gat, agg_sh.at[didx], add=True)

        plsc.subcore_barrier()
        pltpu.sync_copy(
            agg_sh.at[pl.ds(sid * ROWS_PER_TILE, ROWS_PER_TILE)],
            out_hbm.at[cid, pl.ds(sid * ROWS_PER_TILE, ROWS_PER_TILE)])

    return k(hW, eWb, src_p, dst_p)


def kernel(atom_feature, edge_index, bond_feature, distance, node2graph,
           b_factor, Wn, bn, We, be, Wm, bm, Wu, bu, Wo, bo):
    f32 = jnp.float32
    src_p = jnp.concatenate(
        [edge_index[0], jnp.zeros((E_PAD - E,), jnp.int32)])
    dst_p = jnp.concatenate(
        [edge_index[1], jnp.full((E_PAD - E,), N, jnp.int32)])
    bond_p = jnp.concatenate(
        [bond_feature, jnp.zeros((E_PAD - E, D_E_RAW), f32)], axis=0)
    dist_p = jnp.concatenate(
        [distance, jnp.zeros((E_PAD - E,), f32)]).reshape(E_PAD, 1)

    Wm_top = Wm[:, :DH, :]
    Wm_bot = Wm[:, DH:, :]
    Wu_top = Wu[:, :DH, :]
    Wu_bot = Wu[:, DH:, :]
    bm3 = bm.reshape(NB, 1, DH)
    bu3 = bu.reshape(NB, 1, DH)
    bn2 = bn.reshape(1, DH)
    be2 = be.reshape(1, DH)
    bo2 = bo.reshape(1, 1)

    eW = _edge_pre(bond_p, dist_p, We, be2, Wm_bot, bm3)
    h, hW = _node_init(atom_feature, Wn, bn2, Wm_top[0])

    for b in range(NB):
        agg2 = _sc_agg(hW, eW[b], src_p, dst_p)
        if b < NB - 1:
            h, hW = _update(h, agg2, Wu_top[b], Wu_bot[b], bu3[b], Wm_top[b + 1])
        else:
            out = _update_last(h, agg2, Wu_top[b], Wu_bot[b], bu3[b], Wo, bo2)

    return (out.reshape(N), b_factor)
